# baseline (speedup 1.0000x reference)
import functools

import jax
import jax.numpy as jnp
import numpy as np
from jax import lax
from jax.experimental import pallas as pl
from jax.experimental.pallas import tpu as pltpu

_NC = 19
_EPS = float(np.finfo(np.float32).eps)


def _body(x_ref, t_ref, o_ref, acc_ref, *, num_steps, rows):
    step = pl.program_id(0)

    @pl.when(step == 0)
    def _init():
        acc_ref[...] = jnp.zeros_like(acc_ref)

    x = x_ref[0]
    t = t_ref[...]
    cls = lax.broadcasted_iota(jnp.int32, (_NC, rows, 512), 0)
    m = jnp.max(x, axis=0, keepdims=True)
    pred = jnp.min(jnp.where(x == m, cls, _NC), axis=0, keepdims=True)
    npx = rows * 512
    t2 = t.reshape(1, npx)
    p2 = pred.reshape(1, npx)
    cls2 = lax.broadcasted_iota(jnp.int32, (_NC, npx), 0)
    a = (cls2 == t2).astype(jnp.bfloat16)
    b = (cls2 == p2).astype(jnp.bfloat16)
    acc_ref[...] += lax.dot_general(
        a, b, (((1,), (1,)), ((), ())),
        preferred_element_type=jnp.float32)

    @pl.when(step == num_steps - 1)
    def _finalize():
        hist = acc_ref[...]
        r0 = lax.broadcasted_iota(jnp.int32, (_NC, _NC), 0)
        r1 = lax.broadcasted_iota(jnp.int32, (_NC, _NC), 1)
        diag = (r0 == r1).astype(jnp.float32)
        tp = jnp.sum(hist * diag, axis=1)
        sum1 = jnp.sum(hist, axis=1)
        sum0 = jnp.sum(hist, axis=0)
        iou = tp / (sum1 + sum0 - tp + _EPS)
        o_ref[...] = jnp.reshape(jnp.sum(iou) * (100.0 / _NC), (1, 1))


def kernel(input_img, input, target):
    del input_img
    n_b, n_c, h, w = input.shape
    rows = 64
    steps_per_b = h // rows
    num_steps = n_b * steps_per_b

    out = pl.pallas_call(
        functools.partial(_body, num_steps=num_steps, rows=rows),
        grid=(num_steps,),
        in_specs=[
            pl.BlockSpec((1, n_c, rows, w),
                         lambda i: (i // steps_per_b, 0, i % steps_per_b, 0)),
            pl.BlockSpec((1, rows, w),
                         lambda i: (i // steps_per_b, i % steps_per_b, 0)),
        ],
        out_specs=pl.BlockSpec((1, 1), lambda i: (0, 0)),
        out_shape=jax.ShapeDtypeStruct((1, 1), jnp.float32),
        scratch_shapes=[pltpu.VMEM((_NC, _NC), jnp.float32)],
    )(input, target)
    return out.reshape(1)
